# baseline (speedup 1.0000x reference)
import functools

import jax
import jax.numpy as jnp
from jax.experimental import pallas as pl

NUM_TOKENS = 16384
HIDDEN = 2048
EXPERTS = 64
BLOCK = 1024
NSTREAM = 2
BAND = NUM_TOKENS // NSTREAM
STEPS = BAND // BLOCK
OROWS = BLOCK // 128


def _reduce(logits):
    m = jnp.max(logits, axis=1, keepdims=True)
    s = jnp.sum(jnp.exp(logits - m), axis=1, keepdims=True)
    lane = jax.lax.broadcasted_iota(jnp.int32, logits.shape, 1)
    idx = jnp.min(jnp.where(logits == m, lane, EXPERTS), axis=1, keepdims=True)
    return 1.0 / s, idx


def _router_block(x0_ref, x1_ref, w_ref, w0_ref, i0_ref, w1_ref, i1_ref):
    w = w_ref[...]
    for x_ref, wo_ref, io_ref in ((x0_ref, w0_ref, i0_ref),
                                  (x1_ref, w1_ref, i1_ref)):
        logits = jax.lax.dot_general(
            x_ref[...], w, (((1,), (1,)), ((), ())),
            preferred_element_type=jnp.float32)
        wts, idx = _reduce(logits)
        wo_ref[...] = jnp.reshape(wts, (OROWS, 128))
        io_ref[...] = jnp.reshape(idx, (OROWS, 128))


@functools.partial(jax.jit, static_argnames=())
def kernel(hidden_states, W_gate):
    band_out = [jax.ShapeDtypeStruct((BAND // 128, 128), jnp.float32),
                jax.ShapeDtypeStruct((BAND // 128, 128), jnp.int32)]
    w0, i0, w1, i1 = pl.pallas_call(
        _router_block,
        grid=(STEPS,),
        in_specs=[
            pl.BlockSpec((BLOCK, HIDDEN), lambda i: (i, 0)),
            pl.BlockSpec((BLOCK, HIDDEN), lambda i: (i + STEPS, 0)),
            pl.BlockSpec((EXPERTS, HIDDEN), lambda i: (0, 0)),
        ],
        out_specs=[pl.BlockSpec((OROWS, 128), lambda i: (i, 0))] * 4,
        out_shape=band_out + band_out,
    )(hidden_states, hidden_states, W_gate)
    weights = jnp.concatenate([w0, w1], axis=0).reshape(NUM_TOKENS, 1)
    indices = jnp.concatenate([i0, i1], axis=0).reshape(NUM_TOKENS, 1)
    return weights, indices.astype(jnp.int64)


# speedup vs baseline: 1.0694x; 1.0694x over previous
import functools

import jax
import jax.numpy as jnp
from jax.experimental import pallas as pl

NUM_TOKENS = 16384
HIDDEN = 2048
EXPERTS = 64
BLOCK = 2048
STEPS = NUM_TOKENS // BLOCK
OROWS = BLOCK // 128
KHALF = HIDDEN // 2


def _router_block(xl_ref, xr_ref, w_ref, w_out_ref, idx_out_ref):
    w = w_ref[...]
    dims = (((1,), (1,)), ((), ()))
    logits = jax.lax.dot_general(
        xl_ref[...], w[:, :KHALF], dims, preferred_element_type=jnp.float32)
    logits += jax.lax.dot_general(
        xr_ref[...], w[:, KHALF:], dims, preferred_element_type=jnp.float32)
    m = jnp.max(logits, axis=1, keepdims=True)
    s = jnp.sum(jnp.exp(logits - m), axis=1, keepdims=True)
    lane = jax.lax.broadcasted_iota(jnp.int32, logits.shape, 1)
    idx = jnp.min(jnp.where(logits == m, lane, EXPERTS), axis=1, keepdims=True)
    w_out_ref[...] = jnp.reshape(1.0 / s, (OROWS, 128))
    idx_out_ref[...] = jnp.reshape(idx, (OROWS, 128))


@functools.partial(jax.jit, static_argnames=())
def kernel(hidden_states, W_gate):
    weights, indices = pl.pallas_call(
        _router_block,
        grid=(STEPS,),
        in_specs=[
            pl.BlockSpec((BLOCK, KHALF), lambda i: (i, 0)),
            pl.BlockSpec((BLOCK, KHALF), lambda i: (i, 1)),
            pl.BlockSpec((EXPERTS, HIDDEN), lambda i: (0, 0)),
        ],
        out_specs=[
            pl.BlockSpec((OROWS, 128), lambda i: (i, 0)),
            pl.BlockSpec((OROWS, 128), lambda i: (i, 0)),
        ],
        out_shape=[
            jax.ShapeDtypeStruct((NUM_TOKENS // 128, 128), jnp.float32),
            jax.ShapeDtypeStruct((NUM_TOKENS // 128, 128), jnp.int32),
        ],
    )(hidden_states, hidden_states, W_gate)
    return (weights.reshape(NUM_TOKENS, 1),
            indices.reshape(NUM_TOKENS, 1).astype(jnp.int64))
